# baseline (speedup 1.0000x reference)
import functools

import jax
import jax.numpy as jnp
from jax.experimental import pallas as pl

_HID = 768
_BS = 512


def _stats_body(sub_ref, pos_ref, ones_ref, m_ref, r_ref):
    x = sub_ref[...]
    p = pos_ref[...]
    ones = ones_ref[...]
    dims = (((1,), (1,)), ((), ()))
    sum_x = jnp.sum(x, axis=-1, keepdims=True)
    sumsq_x = jnp.sum(x * x, axis=-1, keepdims=True)
    sum_p = jax.lax.dot_general(ones, p, dims,
                                preferred_element_type=jnp.float32)
    sumsq_p = jax.lax.dot_general(ones, p * p, dims,
                                  preferred_element_type=jnp.float32)
    xdotp = jax.lax.dot_general(x, p, dims,
                                preferred_element_type=jnp.float32)
    inv_h = jnp.float32(1.0 / _HID)
    m = (sum_x + sum_p) * inv_h
    e2 = (sumsq_x + 2.0 * xdotp + sumsq_p) * inv_h
    m_ref[...] = m
    r_ref[...] = jax.lax.rsqrt(e2 - m * m + 1e-12)


def _apply_body(sub_ref, pos_ref, m_ref, r_ref, gamma_ref, beta_ref, out_ref):
    x = sub_ref[0]
    p = pos_ref[...]
    m = m_ref[0, 0]
    r = r_ref[0, 0]
    out_ref[0] = (((p + x) - m) * r) * gamma_ref[...] + beta_ref[...]


@jax.jit
def _run(sub2d, table, gamma2d, beta2d):
    S = table.shape[0]
    BN = sub2d.shape[0]
    SB = S // _BS
    ones = jnp.ones((1, _HID), jnp.float32)
    m, r = pl.pallas_call(
        _stats_body,
        out_shape=[
            jax.ShapeDtypeStruct((BN, S), jnp.float32),
            jax.ShapeDtypeStruct((BN, S), jnp.float32),
        ],
    )(sub2d, table, ones)

    sub3d = sub2d.reshape(BN, 1, _HID)
    m4 = m.reshape(BN, SB, _BS, 1)
    r4 = r.reshape(BN, SB, _BS, 1)
    grid = (SB, BN)
    out = pl.pallas_call(
        _apply_body,
        grid=grid,
        in_specs=[
            pl.BlockSpec((1, 1, _HID), lambda i, j: (j, 0, 0)),
            pl.BlockSpec((_BS, _HID), lambda i, j: (i, 0)),
            pl.BlockSpec((1, 1, _BS, 1), lambda i, j: (j, i, 0, 0)),
            pl.BlockSpec((1, 1, _BS, 1), lambda i, j: (j, i, 0, 0)),
            pl.BlockSpec((1, _HID), lambda i, j: (0, 0)),
            pl.BlockSpec((1, _HID), lambda i, j: (0, 0)),
        ],
        out_specs=pl.BlockSpec((1, _BS, _HID), lambda i, j: (j, i, 0)),
        out_shape=jax.ShapeDtypeStruct((BN, S, _HID), jnp.float32),
    )(sub3d, table, m4, r4, gamma2d, beta2d)
    return out


def kernel(sub_goal, seq_length, pos_table, gamma, beta):
    B, N, H = sub_goal.shape
    S = pos_table.shape[0]
    sub2d = sub_goal.reshape(B * N, H)
    out = _run(sub2d, pos_table, gamma.reshape(1, H), beta.reshape(1, H))
    return out.reshape(B, N, S, H)


# speedup vs baseline: 2.0362x; 2.0362x over previous
import functools

import jax
import jax.numpy as jnp
from jax.experimental import pallas as pl
from jax.experimental.pallas import tpu as pltpu

_HID = 768
_BS = 512


def _ln_body(sub_ref, pos_ref, gamma_ref, beta_ref, out_ref):
    x = sub_ref[0]
    p = pos_ref[...]
    e = p + x
    mean = jnp.mean(e, axis=-1, keepdims=True)
    c = e - mean
    var = jnp.mean(c * c, axis=-1, keepdims=True)
    r = jax.lax.rsqrt(var + 1e-12)
    out_ref[0] = (c * r) * gamma_ref[...] + beta_ref[...]


@jax.jit
def _run(sub2d, table, gamma2d, beta2d):
    S = table.shape[0]
    BN = sub2d.shape[0]
    sub3d = sub2d.reshape(BN, 1, _HID)
    grid = (S // _BS, BN)
    out = pl.pallas_call(
        _ln_body,
        grid=grid,
        in_specs=[
            pl.BlockSpec((1, 1, _HID), lambda i, j: (j, 0, 0)),
            pl.BlockSpec((_BS, _HID), lambda i, j: (i, 0)),
            pl.BlockSpec((1, _HID), lambda i, j: (0, 0)),
            pl.BlockSpec((1, _HID), lambda i, j: (0, 0)),
        ],
        out_specs=pl.BlockSpec((1, _BS, _HID), lambda i, j: (j, i, 0)),
        out_shape=jax.ShapeDtypeStruct((BN, S, _HID), jnp.float32),
        compiler_params=pltpu.CompilerParams(
            dimension_semantics=("parallel", "parallel"),
        ),
    )(sub3d, table, gamma2d, beta2d)
    return out


def kernel(sub_goal, seq_length, pos_table, gamma, beta):
    B, N, H = sub_goal.shape
    S = pos_table.shape[0]
    sub2d = sub_goal.reshape(B * N, H)
    out = _run(sub2d, pos_table, gamma.reshape(1, H), beta.reshape(1, H))
    return out.reshape(B, N, S, H)


# speedup vs baseline: 2.6658x; 1.3092x over previous
import functools

import jax
import jax.numpy as jnp
from jax.experimental import pallas as pl
from jax.experimental.pallas import tpu as pltpu

_HID = 768
_BS = 1024


def _ln_body(sub_ref, pos_ref, gamma_ref, beta_ref, out_ref):
    x = sub_ref[0]
    p = pos_ref[...]
    e = p + x
    mean = jnp.mean(e, axis=-1, keepdims=True)
    c = e - mean
    var = jnp.mean(c * c, axis=-1, keepdims=True)
    r = jax.lax.rsqrt(var + 1e-12)
    out_ref[0] = (c * r) * gamma_ref[...] + beta_ref[...]


@jax.jit
def _run(sub2d, table, gamma2d, beta2d):
    S = table.shape[0]
    BN = sub2d.shape[0]
    sub3d = sub2d.reshape(BN, 1, _HID)
    grid = (S // _BS, BN)
    out = pl.pallas_call(
        _ln_body,
        grid=grid,
        in_specs=[
            pl.BlockSpec((1, 1, _HID), lambda i, j: (j, 0, 0)),
            pl.BlockSpec((_BS, _HID), lambda i, j: (i, 0)),
            pl.BlockSpec((1, _HID), lambda i, j: (0, 0)),
            pl.BlockSpec((1, _HID), lambda i, j: (0, 0)),
        ],
        out_specs=pl.BlockSpec((1, _BS, _HID), lambda i, j: (j, i, 0)),
        out_shape=jax.ShapeDtypeStruct((BN, S, _HID), jnp.float32),
        compiler_params=pltpu.CompilerParams(
            dimension_semantics=("parallel", "parallel"),
        ),
    )(sub3d, table, gamma2d, beta2d)
    return out


def kernel(sub_goal, seq_length, pos_table, gamma, beta):
    B, N, H = sub_goal.shape
    S = pos_table.shape[0]
    sub2d = sub_goal.reshape(B * N, H)
    out = _run(sub2d, pos_table, gamma.reshape(1, H), beta.reshape(1, H))
    return out.reshape(B, N, S, H)


# speedup vs baseline: 3.0593x; 1.1476x over previous
import functools

import jax
import jax.numpy as jnp
from jax.experimental import pallas as pl
from jax.experimental.pallas import tpu as pltpu

_HID = 768
_BS = 2048


def _ln_body(sub_ref, pos_ref, gamma_ref, beta_ref, out_ref):
    x = sub_ref[0]
    p = pos_ref[...]
    e = p + x
    mean = jnp.mean(e, axis=-1, keepdims=True)
    c = e - mean
    var = jnp.mean(c * c, axis=-1, keepdims=True)
    r = jax.lax.rsqrt(var + 1e-12)
    out_ref[0] = (c * r) * gamma_ref[...] + beta_ref[...]


@jax.jit
def _run(sub2d, table, gamma2d, beta2d):
    S = table.shape[0]
    BN = sub2d.shape[0]
    sub3d = sub2d.reshape(BN, 1, _HID)
    grid = (S // _BS, BN)
    out = pl.pallas_call(
        _ln_body,
        grid=grid,
        in_specs=[
            pl.BlockSpec((1, 1, _HID), lambda i, j: (j, 0, 0)),
            pl.BlockSpec((_BS, _HID), lambda i, j: (i, 0)),
            pl.BlockSpec((1, _HID), lambda i, j: (0, 0)),
            pl.BlockSpec((1, _HID), lambda i, j: (0, 0)),
        ],
        out_specs=pl.BlockSpec((1, _BS, _HID), lambda i, j: (j, i, 0)),
        out_shape=jax.ShapeDtypeStruct((BN, S, _HID), jnp.float32),
        compiler_params=pltpu.CompilerParams(
            dimension_semantics=("parallel", "parallel"),
        ),
    )(sub3d, table, gamma2d, beta2d)
    return out


def kernel(sub_goal, seq_length, pos_table, gamma, beta):
    B, N, H = sub_goal.shape
    S = pos_table.shape[0]
    sub2d = sub_goal.reshape(B * N, H)
    out = _run(sub2d, pos_table, gamma.reshape(1, H), beta.reshape(1, H))
    return out.reshape(B, N, S, H)


# speedup vs baseline: 3.1115x; 1.0171x over previous
import functools

import jax
import jax.numpy as jnp
from jax.experimental import pallas as pl
from jax.experimental.pallas import tpu as pltpu

_HID = 768
_KN = 2


def _ln_body(sub_ref, pos_ref, gamma_ref, beta_ref, out_ref):
    x = sub_ref[...]
    p = pos_ref[...]
    e = p + x
    mean = jnp.mean(e, axis=-1, keepdims=True)
    c = e - mean
    var = jnp.mean(c * c, axis=-1, keepdims=True)
    r = jax.lax.rsqrt(var + 1e-12)
    out_ref[...] = (c * r) * gamma_ref[...] + beta_ref[...]


@jax.jit
def _run(sub2d, table, gamma2d, beta2d):
    S = table.shape[0]
    BN = sub2d.shape[0]
    sub3d = sub2d.reshape(BN, 1, _HID)
    grid = (BN // _KN,)
    out = pl.pallas_call(
        _ln_body,
        grid=grid,
        in_specs=[
            pl.BlockSpec((_KN, 1, _HID), lambda j: (j, 0, 0)),
            pl.BlockSpec((S, _HID), lambda j: (0, 0)),
            pl.BlockSpec((1, _HID), lambda j: (0, 0)),
            pl.BlockSpec((1, _HID), lambda j: (0, 0)),
        ],
        out_specs=pl.BlockSpec((_KN, S, _HID), lambda j: (j, 0, 0)),
        out_shape=jax.ShapeDtypeStruct((BN, S, _HID), jnp.float32),
        compiler_params=pltpu.CompilerParams(
            dimension_semantics=("parallel",),
        ),
    )(sub3d, table, gamma2d, beta2d)
    return out


def kernel(sub_goal, seq_length, pos_table, gamma, beta):
    B, N, H = sub_goal.shape
    S = pos_table.shape[0]
    sub2d = sub_goal.reshape(B * N, H)
    out = _run(sub2d, pos_table, gamma.reshape(1, H), beta.reshape(1, H))
    return out.reshape(B, N, S, H)


# speedup vs baseline: 3.2157x; 1.0335x over previous
import functools

import jax
import jax.numpy as jnp
from jax.experimental import pallas as pl
from jax.experimental.pallas import tpu as pltpu

_HID = 768
_KN = 2


def _stats_body(sub_ref, pos_ref, ones_ref, m_ref, r_ref):
    x = sub_ref[...]
    p = pos_ref[...]
    ones = ones_ref[...]
    dims = (((1,), (1,)), ((), ()))
    sum_p = jnp.sum(p, axis=-1, keepdims=True)
    sumsq_p = jnp.sum(p * p, axis=-1, keepdims=True)
    sum_x = jax.lax.dot_general(ones, x, dims,
                                preferred_element_type=jnp.float32)
    sumsq_x = jax.lax.dot_general(ones, x * x, dims,
                                  preferred_element_type=jnp.float32)
    pdotx = jax.lax.dot_general(p, x, dims,
                                preferred_element_type=jnp.float32)
    inv_h = jnp.float32(1.0 / _HID)
    m = (sum_p + sum_x) * inv_h
    e2 = (sumsq_p + 2.0 * pdotx + sumsq_x) * inv_h
    m_ref[...] = m
    r_ref[...] = jax.lax.rsqrt(e2 - m * m + 1e-12)


def _apply_body(sub_ref, pos_ref, m_ref, r_ref, gamma_ref, beta_ref, out_ref):
    j = pl.program_id(0)
    p = pos_ref[...]
    mt = m_ref[...]
    rt = r_ref[...]
    g = gamma_ref[...]
    b = beta_ref[...]
    S, BN = mt.shape
    lane = jax.lax.broadcasted_iota(jnp.int32, (S, BN), 1)
    for kn in range(_KN):
        x = sub_ref[kn]
        col = j * _KN + kn
        onehot = lane == col
        m = jnp.sum(jnp.where(onehot, mt, 0.0), axis=1, keepdims=True)
        r = jnp.sum(jnp.where(onehot, rt, 0.0), axis=1, keepdims=True)
        out_ref[kn] = (((p + x) - m) * r) * g + b


@jax.jit
def _run(sub2d, table, gamma2d, beta2d):
    S = table.shape[0]
    BN = sub2d.shape[0]
    ones = jnp.ones((1, _HID), jnp.float32)
    m, r = pl.pallas_call(
        _stats_body,
        out_shape=[
            jax.ShapeDtypeStruct((S, BN), jnp.float32),
            jax.ShapeDtypeStruct((S, BN), jnp.float32),
        ],
    )(sub2d, table, ones)

    sub3d = sub2d.reshape(BN, 1, _HID)
    grid = (BN // _KN,)
    out = pl.pallas_call(
        _apply_body,
        grid=grid,
        in_specs=[
            pl.BlockSpec((_KN, 1, _HID), lambda j: (j, 0, 0)),
            pl.BlockSpec((S, _HID), lambda j: (0, 0)),
            pl.BlockSpec((S, BN), lambda j: (0, 0)),
            pl.BlockSpec((S, BN), lambda j: (0, 0)),
            pl.BlockSpec((1, _HID), lambda j: (0, 0)),
            pl.BlockSpec((1, _HID), lambda j: (0, 0)),
        ],
        out_specs=pl.BlockSpec((_KN, S, _HID), lambda j: (j, 0, 0)),
        out_shape=jax.ShapeDtypeStruct((BN, S, _HID), jnp.float32),
        compiler_params=pltpu.CompilerParams(
            dimension_semantics=("arbitrary",),
        ),
    )(sub3d, table, m, r, gamma2d, beta2d)
    return out


def kernel(sub_goal, seq_length, pos_table, gamma, beta):
    B, N, H = sub_goal.shape
    S = pos_table.shape[0]
    sub2d = sub_goal.reshape(B * N, H)
    out = _run(sub2d, pos_table, gamma.reshape(1, H), beta.reshape(1, H))
    return out.reshape(B, N, S, H)
